# baseline (speedup 1.0000x reference)
import functools

import jax
import jax.numpy as jnp
import numpy as np
from jax import lax
from jax.experimental import pallas as pl
from jax.experimental.pallas import tpu as pltpu
from jax.experimental.pallas import tpu_sc as plsc

NUM_FIELDS = 26
FIELD_DIM = 38462
BATCH = 16384
EMBED_DIM = 16
_OFFSETS = (np.arange(NUM_FIELDS, dtype=np.int32) * FIELD_DIM)

NC = 2
NS = 16
NW = NC * NS
BPW = BATCH // NW
CHUNK = 64
NCHUNK = BPW // CHUNK
ROWS_PER_CHUNK = CHUNK * NUM_FIELDS
LIN_ROWS = BPW * NUM_FIELDS


def _fm_body(xo_hbm, w_hbm, out_hbm,
             idxv0, idxv1, rowsv0, rowsv1, tbuf, outv,
             sem_w0, sem_w1):
    wid = lax.axis_index("s") * NC + lax.axis_index("c")
    iota = lax.iota(jnp.int32, 16)

    idxv = (idxv0, idxv1)
    rowsv = (rowsv0, rowsv1)
    sem_w = (sem_w0, sem_w1)
    base = wid * BPW * NUM_FIELDS

    def fire(c, slot):
        pltpu.sync_copy(
            xo_hbm.at[pl.ds(base + c * ROWS_PER_CHUNK, ROWS_PER_CHUNK)],
            idxv[slot])
        return pltpu.async_copy(w_hbm.at[idxv[slot]], rowsv[slot], sem_w[slot])

    inflight = fire(0, 0)
    for c in range(NCHUNK):
        slot = c % 2
        cw = inflight
        if c + 1 < NCHUNK:
            inflight = fire(c + 1, 1 - slot)
        cw.wait()
        rv = rowsv[slot]

        def bi_body(bi, _):
            p0 = bi * NUM_FIELDS
            acc_s = jnp.zeros((16,), jnp.float32)
            acc_q = jnp.zeros((16,), jnp.float32)
            for f in range(NUM_FIELDS):
                v = rv[p0 + f]
                acc_s = acc_s + v
                acc_q = acc_q + v * v
            tbuf[pl.ds(bi * 16, 16)] = acc_s * acc_s - acc_q
            return 0

        lax.fori_loop(0, CHUNK, bi_body, 0, unroll=False)

        def q_body(q, _):
            ti0 = (q * 16 + iota) * 16
            fm = jnp.zeros((16,), jnp.float32)
            for j in range(16):
                fm = fm + plsc.load_gather(tbuf, [ti0 + j])
            outv[pl.ds(c * CHUNK + q * 16, 16)] = 0.5 * fm
            return 0

        lax.fori_loop(0, CHUNK // 16, q_body, 0, unroll=False)

    pltpu.sync_copy(outv, out_hbm.at[pl.ds(wid * BPW, BPW)])


def _lin_body(xo_hbm, wlin_hbm, fm_hbm, blin_hbm, out_hbm,
              idxv, linv, fmv, outv, bv, sem_l):
    wid = lax.axis_index("s") * NC + lax.axis_index("c")
    pltpu.sync_copy(blin_hbm, bv)
    bvec = bv[...]
    iota = lax.iota(jnp.int32, 16)
    base = wid * LIN_ROWS

    pltpu.sync_copy(xo_hbm.at[pl.ds(base, LIN_ROWS)], idxv)
    cl = pltpu.async_copy(wlin_hbm.at[idxv], linv, sem_l)
    pltpu.sync_copy(fm_hbm.at[pl.ds(wid * BPW, BPW)], fmv)
    cl.wait()

    def q_body(q, _):
        bi_v = q * 16 + iota
        pv0 = bi_v * NUM_FIELDS
        lin = jnp.zeros((16,), jnp.float32)
        for f in range(NUM_FIELDS):
            lin = lin + plsc.load_gather(linv, [pv0 + f])
        outv[pl.ds(q * 16, 16)] = lin + fmv[pl.ds(q * 16, 16)] + bvec
        return 0

    lax.fori_loop(0, BPW // 16, q_body, 0, unroll=False)
    pltpu.sync_copy(outv, out_hbm.at[pl.ds(wid * BPW, BPW)])


@jax.jit
def kernel(x, W, w_lin, b_lin):
    xo = (x + jnp.asarray(_OFFSETS)[None, :]).reshape(-1)
    blin16 = jnp.broadcast_to(b_lin.astype(jnp.float32), (16,))
    wlin_flat = w_lin.reshape(-1)

    mesh = plsc.VectorSubcoreMesh(core_axis_name="c", subcore_axis_name="s")
    cparams = pltpu.CompilerParams(
        needs_layout_passes=False, use_tc_tiling_on_sc=False)

    fm_kernel = pl.kernel(
        _fm_body,
        out_type=jax.ShapeDtypeStruct((BATCH,), jnp.float32),
        mesh=mesh,
        compiler_params=cparams,
        scratch_types=[
            pltpu.VMEM((ROWS_PER_CHUNK,), jnp.int32),
            pltpu.VMEM((ROWS_PER_CHUNK,), jnp.int32),
            pltpu.VMEM((ROWS_PER_CHUNK, EMBED_DIM), jnp.float32),
            pltpu.VMEM((ROWS_PER_CHUNK, EMBED_DIM), jnp.float32),
            pltpu.VMEM((CHUNK * 16,), jnp.float32),
            pltpu.VMEM((BPW,), jnp.float32),
            pltpu.SemaphoreType.DMA,
            pltpu.SemaphoreType.DMA,
        ],
    )
    fm_part = fm_kernel(xo, W)

    lin_kernel = pl.kernel(
        _lin_body,
        out_type=jax.ShapeDtypeStruct((BATCH,), jnp.float32),
        mesh=mesh,
        compiler_params=cparams,
        scratch_types=[
            pltpu.VMEM((LIN_ROWS,), jnp.int32),
            pltpu.VMEM((LIN_ROWS,), jnp.float32),
            pltpu.VMEM((BPW,), jnp.float32),
            pltpu.VMEM((BPW,), jnp.float32),
            pltpu.VMEM((16,), jnp.float32),
            pltpu.SemaphoreType.DMA,
        ],
    )
    return lin_kernel(xo, wlin_flat, fm_part, blin16)
